# Initial kernel scaffold; baseline (speedup 1.0000x reference)
import jax
import jax.numpy as jnp
from jax.experimental import pallas as pl


def kernel(x):
    raise NotImplementedError("write your pallas kernel here")


# speedup vs baseline: 41.3616x; 41.3616x over previous
import functools

import jax
import jax.numpy as jnp
from jax.experimental import pallas as pl
from jax.experimental.pallas import tpu as pltpu

B = 32
V = 1_000_000
CHUNK = 500
NCHUNK = V // CHUNK
NCAND = 64
TOPK = 50
INV_TEMP = 1.25
KEY_HI = 0
KEY_LO = 42


def _chunkmax_kernel(x_ref, out_ref):
    out_ref[0, 0, :] = jnp.max(x_ref[0], axis=1)


def _phase_a(x3):
    blk = 400
    nb = NCHUNK // blk
    out = pl.pallas_call(
        _chunkmax_kernel,
        grid=(B, nb),
        in_specs=[pl.BlockSpec((1, blk, CHUNK), lambda r, b: (r, b, 0))],
        out_specs=pl.BlockSpec((1, 1, blk), lambda r, b: (r * nb + b, 0, 0)),
        out_shape=jax.ShapeDtypeStruct((B * nb, 1, blk), jnp.float32),
    )(x3)
    return out.reshape(B, NCHUNK)


def _select_kernel(cm_ref, idx_ref):
    v = cm_ref[...]
    lanes = jax.lax.broadcasted_iota(jnp.int32, (B, NCHUNK), 1)
    for i in range(NCAND):
        m = jnp.argmax(v, axis=1).astype(jnp.int32)
        idx_ref[:, i : i + 1] = m[:, None]
        v = jnp.where(lanes == m[:, None], -jnp.inf, v)


def _phase_b(cm):
    return pl.pallas_call(
        _select_kernel,
        grid=(1,),
        in_specs=[pl.BlockSpec((B, NCHUNK), lambda i: (0, 0))],
        out_specs=pl.BlockSpec((B, NCAND), lambda i: (0, 0)),
        out_shape=jax.ShapeDtypeStruct((B, NCAND), jnp.int32),
    )(cm)


def _rotl(x, d):
    return (x << jnp.uint32(d)) | (x >> jnp.uint32(32 - d))


def _gumbel_bits(flat_u32):
    k0, k1 = jnp.uint32(KEY_HI), jnp.uint32(KEY_LO)
    ks2 = jnp.uint32(KEY_HI ^ KEY_LO ^ 0x1BD11BDA)
    ks = (k0, k1, ks2)
    x0 = jnp.zeros_like(flat_u32) + ks[0]
    x1 = flat_u32 + ks[1]
    rot = ((13, 15, 26, 6), (17, 29, 16, 24))
    for g in range(5):
        for r in rot[g % 2]:
            x0 = x0 + x1
            x1 = _rotl(x1, r)
            x1 = x1 ^ x0
        x0 = x0 + ks[(g + 1) % 3]
        x1 = x1 + ks[(g + 2) % 3] + jnp.uint32(g + 1)
    bits = x0 ^ x1
    fb = pltpu.bitcast((bits >> jnp.uint32(9)) | jnp.uint32(0x3F800000),
                       jnp.float32)
    tiny = jnp.float32(jnp.finfo(jnp.float32).tiny)
    u01 = fb - jnp.float32(1.0)
    u = u01 * (jnp.float32(1.0) - tiny) + tiny
    u = jnp.maximum(tiny, u)
    return -jnp.log(-jnp.log(u))


def _sample_kernel(idx_sref, x_ref, out_ref, buf, score, colb, sems):
    r = pl.program_id(0)
    copies = []
    for i in range(NCAND):
        c = idx_sref[r * NCAND + i]
        cp = pltpu.make_async_copy(x_ref.at[r, c], buf.at[i], sems.at[i])
        cp.start()
        copies.append(cp)
    lane = jax.lax.broadcasted_iota(jnp.int32, (1, CHUNK), 1)
    for i in range(NCAND):
        copies[i].wait()
        c = idx_sref[r * NCAND + i]
        col = c * CHUNK + lane
        colb[i : i + 1, :] = col
        flat = pltpu.bitcast(r * V + col, jnp.uint32)
        g = _gumbel_bits(flat)
        score[i : i + 1, :] = buf[i : i + 1, :] * jnp.float32(INV_TEMP) + g
    w = buf[...]
    removed = jnp.int32(0)
    t = jnp.float32(jnp.inf)
    for _ in range(TOPK):
        m = jnp.max(w)
        cnt = jnp.sum(jnp.where(w == m, 1, 0).astype(jnp.int32))
        t = jnp.where(removed < TOPK, m, t)
        w = jnp.where(w == m, -jnp.inf, w)
        removed = removed + cnt
    keep = buf[...] >= t
    s = jnp.where(keep, score[...], -jnp.inf)
    best = jnp.max(s)
    sample = jnp.max(jnp.where(s == best, colb[...], jnp.int32(-1)))
    out_ref[0, 0, :] = jnp.full((128,), sample, jnp.int32)


def _phase_cd(x3, idx):
    grid_spec = pltpu.PrefetchScalarGridSpec(
        num_scalar_prefetch=1,
        grid=(B,),
        in_specs=[pl.BlockSpec(memory_space=pl.MemorySpace.ANY)],
        out_specs=pl.BlockSpec((1, 1, 128), lambda r, idx_ref: (r, 0, 0)),
        scratch_shapes=[
            pltpu.VMEM((NCAND, CHUNK), jnp.float32),
            pltpu.VMEM((NCAND, CHUNK), jnp.float32),
            pltpu.VMEM((NCAND, CHUNK), jnp.int32),
            pltpu.SemaphoreType.DMA((NCAND,)),
        ],
    )
    out = pl.pallas_call(
        _sample_kernel,
        grid_spec=grid_spec,
        out_shape=jax.ShapeDtypeStruct((B, 1, 128), jnp.int32),
    )(idx.reshape(-1), x3)
    return out[:, 0, :1]


@jax.jit
def kernel(x):
    x3 = x.reshape(B, NCHUNK, CHUNK)
    cm = _phase_a(x3)
    idx = _phase_b(cm)
    return _phase_cd(x3, idx)


# speedup vs baseline: 69.1945x; 1.6729x over previous
import functools

import jax
import jax.numpy as jnp
from jax.experimental import pallas as pl
from jax.experimental.pallas import tpu as pltpu

B = 32
V = 1_000_000
CHUNK = 500
NCHUNK = V // CHUNK
NCAND = 64
TOPK = 50
INV_TEMP = 1.25
KEY_HI = 0
KEY_LO = 42


def _chunkmax_kernel(x_ref, out_ref):
    out_ref[0, 0, :] = jnp.max(x_ref[0], axis=1)


def _phase_a(x3):
    blk = 400
    nb = NCHUNK // blk
    out = pl.pallas_call(
        _chunkmax_kernel,
        grid=(B, nb),
        in_specs=[pl.BlockSpec((1, blk, CHUNK), lambda r, b: (r, b, 0))],
        out_specs=pl.BlockSpec((1, 1, blk), lambda r, b: (r * nb + b, 0, 0)),
        out_shape=jax.ShapeDtypeStruct((B * nb, 1, blk), jnp.float32),
    )(x3)
    return out.reshape(B, NCHUNK)


def _select_kernel(cm_ref, idx_ref):
    v = cm_ref[...]
    lanes = jax.lax.broadcasted_iota(jnp.int32, (B, NCHUNK), 1)
    for i in range(NCAND):
        m = jnp.argmax(v, axis=1).astype(jnp.int32)
        idx_ref[:, i : i + 1] = m[:, None]
        v = jnp.where(lanes == m[:, None], -jnp.inf, v)


def _phase_b(cm):
    return pl.pallas_call(
        _select_kernel,
        grid=(1,),
        in_specs=[pl.BlockSpec((B, NCHUNK), lambda i: (0, 0))],
        out_specs=pl.BlockSpec((B, NCAND), lambda i: (0, 0)),
        out_shape=jax.ShapeDtypeStruct((B, NCAND), jnp.int32),
    )(cm)


def _rotl(x, d):
    return (x << jnp.uint32(d)) | (x >> jnp.uint32(32 - d))


def _gumbel_bits(flat_u32):
    k0, k1 = jnp.uint32(KEY_HI), jnp.uint32(KEY_LO)
    ks2 = jnp.uint32(KEY_HI ^ KEY_LO ^ 0x1BD11BDA)
    ks = (k0, k1, ks2)
    x0 = jnp.zeros_like(flat_u32) + ks[0]
    x1 = flat_u32 + ks[1]
    rot = ((13, 15, 26, 6), (17, 29, 16, 24))
    for g in range(5):
        for r in rot[g % 2]:
            x0 = x0 + x1
            x1 = _rotl(x1, r)
            x1 = x1 ^ x0
        x0 = x0 + ks[(g + 1) % 3]
        x1 = x1 + ks[(g + 2) % 3] + jnp.uint32(g + 1)
    bits = x0 ^ x1
    fb = pltpu.bitcast((bits >> jnp.uint32(9)) | jnp.uint32(0x3F800000),
                       jnp.float32)
    tiny = jnp.float32(jnp.finfo(jnp.float32).tiny)
    u01 = fb - jnp.float32(1.0)
    u = u01 * (jnp.float32(1.0) - tiny) + tiny
    u = jnp.maximum(tiny, u)
    return -jnp.log(-jnp.log(u))


def _gather_kernel(idx_sref, x_ref, out_ref, sems):
    r = pl.program_id(0)
    copies = []
    for i in range(NCAND):
        c = idx_sref[r * NCAND + i]
        cp = pltpu.make_async_copy(x_ref.at[r, c], out_ref.at[0, i], sems.at[i])
        cp.start()
        copies.append(cp)
    for cp in copies:
        cp.wait()


def _phase_c(x3, idx):
    grid_spec = pltpu.PrefetchScalarGridSpec(
        num_scalar_prefetch=1,
        grid=(B,),
        in_specs=[pl.BlockSpec(memory_space=pl.MemorySpace.ANY)],
        out_specs=pl.BlockSpec((1, NCAND, CHUNK), lambda r, idx_ref: (r, 0, 0)),
        scratch_shapes=[pltpu.SemaphoreType.DMA((NCAND,))],
    )
    return pl.pallas_call(
        _gather_kernel,
        grid_spec=grid_spec,
        out_shape=jax.ShapeDtypeStruct((B, NCAND, CHUNK), jnp.float32),
    )(idx.reshape(-1), x3)


def _finish_kernel(g_ref, idx_ref, out_ref):
    v = g_ref[...]
    cols = (idx_ref[...][:, :, None] * CHUNK
            + jax.lax.broadcasted_iota(jnp.int32, (B, NCAND, CHUNK), 2))
    rows = jax.lax.broadcasted_iota(jnp.int32, (B, NCAND, CHUNK), 0)
    flat = pltpu.bitcast(rows * V + cols, jnp.uint32)
    gum = _gumbel_bits(flat)
    score = v * jnp.float32(INV_TEMP) + gum
    w = v
    removed = jnp.zeros((B, 1, 1), jnp.int32)
    t = jnp.full((B, 1, 1), jnp.inf, jnp.float32)
    for _ in range(TOPK):
        m = jnp.max(jnp.max(w, axis=2), axis=1)[:, None, None]
        hit = w == m
        cnt = jnp.sum(jnp.sum(hit.astype(jnp.int32), axis=2), axis=1)
        t = jnp.where(removed < TOPK, m, t)
        w = jnp.where(hit, -jnp.inf, w)
        removed = removed + cnt[:, None, None]
    s = jnp.where(v >= t, score, -jnp.inf)
    best = jnp.max(jnp.max(s, axis=2), axis=1)[:, None, None]
    sample = jnp.max(jnp.max(jnp.where(s == best, cols, -1), axis=2), axis=1)
    out_ref[...] = jnp.broadcast_to(sample[:, None, None], (B, 1, 128))


def _phase_d(g, idx):
    out = pl.pallas_call(
        _finish_kernel,
        grid=(1,),
        in_specs=[
            pl.BlockSpec((B, NCAND, CHUNK), lambda i: (0, 0, 0)),
            pl.BlockSpec((B, NCAND), lambda i: (0, 0)),
        ],
        out_specs=pl.BlockSpec((B, 1, 128), lambda i: (0, 0, 0)),
        out_shape=jax.ShapeDtypeStruct((B, 1, 128), jnp.int32),
    )(g, idx)
    return out[:, 0, :1]


@jax.jit
def kernel(x):
    x3 = x.reshape(B, NCHUNK, CHUNK)
    cm = _phase_a(x3)
    idx = _phase_b(cm)
    g = _phase_c(x3, idx)
    return _phase_d(g, idx)
